# baseline (speedup 1.0000x reference)
import functools

import jax
import jax.numpy as jnp
from jax import lax
from jax.experimental import pallas as pl
from jax.experimental.pallas import tpu as pltpu
from jax.experimental.pallas import tpu_sc as plsc

_D = 768
_SEQ = 8192
_BATCH = 4
_NC = 2
_NS = 16
_NW = _NC * _NS
_SEQ_PER_W = _SEQ // _NW
_CHUNK_ROWS = 64
_N_CHUNKS = _SEQ_PER_W // _CHUNK_ROWS
_CH_WORDS = _CHUNK_ROWS * _D
_WORDS_PER_BATCH = _SEQ * _D


def _sc_body(we_hbm, pe_hbm, out_hbm, pbuf, wbuf):
    wid = lax.axis_index("s") * _NC + lax.axis_index("c")
    base_row = wid * _SEQ_PER_W
    for c in range(_N_CHUNKS):
        pe_off = pl.multiple_of((base_row + c * _CHUNK_ROWS) * _D, 8)
        pltpu.sync_copy(pe_hbm.at[pl.ds(pe_off, _CH_WORDS)], pbuf)
        for b in range(_BATCH):
            we_off = pl.multiple_of(b * _WORDS_PER_BATCH + pe_off, 8)
            pltpu.sync_copy(we_hbm.at[pl.ds(we_off, _CH_WORDS)], wbuf)

            @plsc.parallel_loop(0, _CH_WORDS, 16, unroll=8)
            def _add(j):
                wbuf[pl.ds(j, 16)] = wbuf[pl.ds(j, 16)] + pbuf[pl.ds(j, 16)]

            pltpu.sync_copy(wbuf, out_hbm.at[pl.ds(we_off, _CH_WORDS)])


_sc_add = functools.partial(
    pl.kernel,
    out_type=jax.ShapeDtypeStruct((_BATCH * _WORDS_PER_BATCH,), jnp.float32),
    mesh=plsc.VectorSubcoreMesh(core_axis_name="c", subcore_axis_name="s"),
    scratch_types=[
        pltpu.VMEM((_CH_WORDS,), jnp.float32),
        pltpu.VMEM((_CH_WORDS,), jnp.float32),
    ],
)(_sc_body)


def kernel(word_embeddings, pe):
    batch, seq_len, d_model = word_embeddings.shape
    out_flat = _sc_add(word_embeddings.reshape(-1), pe.reshape(-1))
    return out_flat.reshape(batch, seq_len, d_model)


# speedup vs baseline: 1.1873x; 1.1873x over previous
import functools

import jax
import jax.numpy as jnp
from jax import lax
from jax.experimental import pallas as pl
from jax.experimental.pallas import tpu as pltpu
from jax.experimental.pallas import tpu_sc as plsc

_D = 768
_SEQ = 8192
_BATCH = 4
_NC = 2
_NS = 16
_NW = _NC * _NS
_SEQ_PER_W = _SEQ // _NW
_CHUNK_ROWS = 32
_N_CHUNKS = _SEQ_PER_W // _CHUNK_ROWS
_CH_WORDS = _CHUNK_ROWS * _D
_WORDS_PER_BATCH = _SEQ * _D
_N_UNITS = _N_CHUNKS * _BATCH


def _sc_body(we_hbm, pe_hbm, out_hbm,
             pb0, pb1, wb0, wb1, wb2,
             spe0, spe1, swe0, swe1, swe2, so0, so1, so2):
    pbufs, pe_sems = (pb0, pb1), (spe0, spe1)
    wbufs, we_sems = (wb0, wb1, wb2), (swe0, swe1, swe2)
    out_sems = (so0, so1, so2)
    wid = lax.axis_index("s") * _NC + lax.axis_index("c")
    base_off = wid * _SEQ_PER_W * _D

    def pe_off(c):
        return pl.multiple_of(base_off + c * _CH_WORDS, 8)

    def we_off(u):
        c, b = u // _BATCH, u % _BATCH
        return pl.multiple_of(b * _WORDS_PER_BATCH + base_off + c * _CH_WORDS, 8)

    def issue_pe(c):
        return pltpu.async_copy(
            pe_hbm.at[pl.ds(pe_off(c), _CH_WORDS)], pbufs[c % 2], pe_sems[c % 2])

    def issue_we(u):
        return pltpu.async_copy(
            we_hbm.at[pl.ds(we_off(u), _CH_WORDS)], wbufs[u % 3], we_sems[u % 3])

    def issue_out(u):
        return pltpu.async_copy(
            wbufs[u % 3], out_hbm.at[pl.ds(we_off(u), _CH_WORDS)], out_sems[u % 3])

    pe_cp = [None, None]
    we_cp = [None, None, None]
    out_cp = [None, None, None]
    pe_cp[0] = issue_pe(0)
    we_cp[0] = issue_we(0)
    pe_waited = [False] * _N_CHUNKS

    for u in range(_N_UNITS):
        c, b = u // _BATCH, u % _BATCH
        if b == 0 and c + 1 < _N_CHUNKS:
            pe_cp[(c + 1) % 2] = issue_pe(c + 1)
        if u + 1 < _N_UNITS:
            if u - 2 >= 0:
                out_cp[(u + 1) % 3].wait()
            we_cp[(u + 1) % 3] = issue_we(u + 1)
        if not pe_waited[c]:
            pe_cp[c % 2].wait()
            pe_waited[c] = True
        we_cp[u % 3].wait()
        wbuf, pbuf = wbufs[u % 3], pbufs[c % 2]

        @plsc.parallel_loop(0, _CH_WORDS, 16, unroll=8)
        def _add(j):
            wbuf[pl.ds(j, 16)] = wbuf[pl.ds(j, 16)] + pbuf[pl.ds(j, 16)]

        out_cp[u % 3] = issue_out(u)

    for u in range(_N_UNITS - 3, _N_UNITS):
        out_cp[u % 3].wait()


_sc_add = functools.partial(
    pl.kernel,
    out_type=jax.ShapeDtypeStruct((_BATCH * _WORDS_PER_BATCH,), jnp.float32),
    mesh=plsc.VectorSubcoreMesh(core_axis_name="c", subcore_axis_name="s"),
    scratch_types=[
        pltpu.VMEM((_CH_WORDS,), jnp.float32),
        pltpu.VMEM((_CH_WORDS,), jnp.float32),
        pltpu.VMEM((_CH_WORDS,), jnp.float32),
        pltpu.VMEM((_CH_WORDS,), jnp.float32),
        pltpu.VMEM((_CH_WORDS,), jnp.float32),
        pltpu.SemaphoreType.DMA,
        pltpu.SemaphoreType.DMA,
        pltpu.SemaphoreType.DMA,
        pltpu.SemaphoreType.DMA,
        pltpu.SemaphoreType.DMA,
        pltpu.SemaphoreType.DMA,
        pltpu.SemaphoreType.DMA,
        pltpu.SemaphoreType.DMA,
    ],
)(_sc_body)


def kernel(word_embeddings, pe):
    batch, seq_len, d_model = word_embeddings.shape
    out_flat = _sc_add(word_embeddings.reshape(-1), pe.reshape(-1))
    return out_flat.reshape(batch, seq_len, d_model)


# speedup vs baseline: 3.6214x; 3.0502x over previous
import functools

import jax
import jax.numpy as jnp
from jax import lax
from jax.experimental import pallas as pl
from jax.experimental.pallas import tpu as pltpu
from jax.experimental.pallas import tpu_sc as plsc

_D = 768
_SEQ = 8192
_BATCH = 4
_NC = 2
_NS = 16
_NW = _NC * _NS
_SEQ_PER_W = _SEQ // _NW
_CHUNK_ROWS = 32
_N_CHUNKS = _SEQ_PER_W // _CHUNK_ROWS
_VECS_PER_ROW = _D // 16
_N_UNITS = _N_CHUNKS * _BATCH


def _sc_body(we_hbm, pe_hbm, out_hbm,
             pb0, pb1, wb0, wb1, wb2,
             spe0, spe1, swe0, swe1, swe2, so0, so1, so2):
    pbufs, pe_sems = (pb0, pb1), (spe0, spe1)
    wbufs, we_sems = (wb0, wb1, wb2), (swe0, swe1, swe2)
    out_sems = (so0, so1, so2)
    wid = lax.axis_index("s") * _NC + lax.axis_index("c")
    base_row = wid * _SEQ_PER_W

    def row0(c):
        return pl.multiple_of(base_row + c * _CHUNK_ROWS, 8)

    def issue_pe(c):
        return pltpu.async_copy(
            pe_hbm.at[pl.ds(row0(c), _CHUNK_ROWS), :], pbufs[c % 2], pe_sems[c % 2])

    def issue_we(u):
        c, b = u // _BATCH, u % _BATCH
        return pltpu.async_copy(
            we_hbm.at[b, pl.ds(row0(c), _CHUNK_ROWS), :], wbufs[u % 3], we_sems[u % 3])

    def issue_out(u):
        c, b = u // _BATCH, u % _BATCH
        return pltpu.async_copy(
            wbufs[u % 3], out_hbm.at[b, pl.ds(row0(c), _CHUNK_ROWS), :], out_sems[u % 3])

    pe_cp = [None, None]
    we_cp = [None, None, None]
    out_cp = [None, None, None]
    pe_cp[0] = issue_pe(0)
    we_cp[0] = issue_we(0)
    pe_waited = [False] * _N_CHUNKS

    for u in range(_N_UNITS):
        c, b = u // _BATCH, u % _BATCH
        if b == 0 and c + 1 < _N_CHUNKS:
            pe_cp[(c + 1) % 2] = issue_pe(c + 1)
        if u + 1 < _N_UNITS:
            if u - 2 >= 0:
                out_cp[(u + 1) % 3].wait()
            we_cp[(u + 1) % 3] = issue_we(u + 1)
        if not pe_waited[c]:
            pe_cp[c % 2].wait()
            pe_waited[c] = True
        we_cp[u % 3].wait()
        wbuf, pbuf = wbufs[u % 3], pbufs[c % 2]

        @plsc.parallel_loop(0, _CHUNK_ROWS * _VECS_PER_ROW, 1, unroll=8)
        def _add(j):
            i = j // _VECS_PER_ROW
            v = (j - i * _VECS_PER_ROW) * 16
            wbuf[i, pl.ds(v, 16)] = wbuf[i, pl.ds(v, 16)] + pbuf[i, pl.ds(v, 16)]

        out_cp[u % 3] = issue_out(u)

    for u in range(_N_UNITS - 3, _N_UNITS):
        out_cp[u % 3].wait()


_sc_add = functools.partial(
    pl.kernel,
    out_type=jax.ShapeDtypeStruct((_BATCH, _SEQ, _D), jnp.float32),
    mesh=plsc.VectorSubcoreMesh(core_axis_name="c", subcore_axis_name="s"),
    scratch_types=[
        pltpu.VMEM((_CHUNK_ROWS, _D), jnp.float32),
        pltpu.VMEM((_CHUNK_ROWS, _D), jnp.float32),
        pltpu.VMEM((_CHUNK_ROWS, _D), jnp.float32),
        pltpu.VMEM((_CHUNK_ROWS, _D), jnp.float32),
        pltpu.VMEM((_CHUNK_ROWS, _D), jnp.float32),
        pltpu.SemaphoreType.DMA,
        pltpu.SemaphoreType.DMA,
        pltpu.SemaphoreType.DMA,
        pltpu.SemaphoreType.DMA,
        pltpu.SemaphoreType.DMA,
        pltpu.SemaphoreType.DMA,
        pltpu.SemaphoreType.DMA,
        pltpu.SemaphoreType.DMA,
    ],
)(_sc_body)


def kernel(word_embeddings, pe):
    return _sc_add(word_embeddings, pe)
